# baseline (speedup 1.0000x reference)
import functools

import jax
import jax.numpy as jnp
from jax import lax
from jax.experimental import pallas as pl
from jax.experimental.pallas import tpu as pltpu
from jax.experimental.pallas import tpu_sc as plsc

N = 10000
E = 160000
F = 128
NB = 8
SHD = 9
NSPEC = 10
CORR = 3
AVG = 16.0

NC = 2
NS = 16
NW = NC * NS
EPT = 5120
E_PAD = NW * EPT
BATCH = 128
NBATCH = EPT // BATCH
N_PAD = 10240
RPT = N_PAD // NS

BE = 512
BN = 1000


def _gather_body(nf_hbm, snd_hbm, h_hbm, idx_v, rows_v, sem):
    c = lax.axis_index("c")
    s = lax.axis_index("s")
    base = (c * NS + s) * EPT
    pltpu.sync_copy(snd_hbm.at[pl.ds(base, EPT)], idx_v)

    def body(j, carry):
        off = pl.multiple_of(j * BATCH, BATCH)
        pltpu.async_copy(nf_hbm.at[idx_v.at[pl.ds(off, BATCH)]], rows_v, sem).wait()
        pltpu.sync_copy(rows_v, h_hbm.at[pl.ds(base + off, BATCH)])
        return carry

    lax.fori_loop(0, NBATCH, body, 0)


@functools.cache
def _gather():
    mesh = plsc.VectorSubcoreMesh(core_axis_name="c", subcore_axis_name="s")
    return pl.kernel(
        _gather_body,
        out_type=jax.ShapeDtypeStruct((E_PAD, F), jnp.float32),
        mesh=mesh,
        scratch_types=[
            pltpu.VMEM((EPT,), jnp.int32),
            pltpu.VMEM((BATCH, F), jnp.float32),
            pltpu.SemaphoreType.DMA,
        ],
    )


def _scatter_body(m_hbm, recv3_hbm, zeros_hbm, out_hbm, ridx_v, m_v, acc_sh):
    c = lax.axis_index("c")
    s = lax.axis_index("s")
    wid = c * NS + s
    base = wid * EPT
    row0 = s * RPT
    pltpu.sync_copy(zeros_hbm.at[pl.ds(row0, RPT)], acc_sh.at[pl.ds(row0, RPT)])
    pltpu.sync_copy(recv3_hbm.at[wid], ridx_v)
    plsc.subcore_barrier()

    def body(j, carry):
        off = pl.multiple_of(j * BATCH, BATCH)
        pltpu.sync_copy(m_hbm.at[pl.ds(base + off, BATCH)], m_v)
        pltpu.sync_copy(m_v, acc_sh.at[ridx_v.at[j]], add=True)
        return carry

    lax.fori_loop(0, NBATCH, body, 0)
    plsc.subcore_barrier()
    pltpu.sync_copy(acc_sh.at[pl.ds(row0, RPT)], out_hbm.at[c, pl.ds(row0, RPT)])


@functools.cache
def _scatter():
    mesh = plsc.VectorSubcoreMesh(core_axis_name="c", subcore_axis_name="s")
    return pl.kernel(
        _scatter_body,
        out_type=jax.ShapeDtypeStruct((NC, N_PAD, F), jnp.float32),
        mesh=mesh,
        scratch_types=[
            pltpu.VMEM((NBATCH, BATCH), jnp.int32),
            pltpu.VMEM((BATCH, F), jnp.float32),
            pltpu.VMEM_SHARED((N_PAD, F), jnp.float32),
        ],
    )


def _edge_body(vec_ref, rad_ref, h_ref, wr1_ref, br1_ref, wr2_ref, br2_ref,
               wlin_ref, m_ref):
    v = vec_ref[...]
    r = jnp.sqrt(jnp.sum(v * v, axis=1, keepdims=True)) + 1e-8
    u = v / r
    x, y, z = u[:, 0:1], u[:, 1:2], u[:, 2:3]
    yh = jnp.concatenate([
        jnp.ones_like(x), x, y, z,
        x * y, y * z, 3.0 * z * z - 1.0, x * z, x * x - y * y,
    ], axis=1)
    rh = rad_ref[...] @ wr1_ref[...] + br1_ref[...]
    rh = rh * jax.nn.sigmoid(rh)
    rw = rh @ wr2_ref[...] + br2_ref[...]
    cc = yh * rw
    h = h_ref[...]
    x = jnp.concatenate([h * cc[:, lm:lm + 1] for lm in range(SHD)], axis=1)
    acc = jnp.dot(x, wlin_ref[...], preferred_element_type=jnp.float32)
    m_ref[...] = acc * (1.0 / jnp.sqrt(AVG))


def _edge_call(vec_p, rad_p, h, wr1, br1, wr2, br2, wlin3):
    grid = E_PAD // BE
    return pl.pallas_call(
        _edge_body,
        grid=(grid,),
        in_specs=[
            pl.BlockSpec((BE, 3), lambda i: (i, 0)),
            pl.BlockSpec((BE, NB), lambda i: (i, 0)),
            pl.BlockSpec((BE, F), lambda i: (i, 0)),
            pl.BlockSpec((NB, 64), lambda i: (0, 0)),
            pl.BlockSpec((1, 64), lambda i: (0, 0)),
            pl.BlockSpec((64, SHD), lambda i: (0, 0)),
            pl.BlockSpec((1, SHD), lambda i: (0, 0)),
            pl.BlockSpec((SHD * F, F), lambda i: (0, 0)),
        ],
        out_specs=pl.BlockSpec((BE, F), lambda i: (i, 0)),
        out_shape=jax.ShapeDtypeStruct((E_PAD, F), jnp.float32),
    )(vec_p, rad_p, h, wr1, br1, wr2, br2, wlin3)


def _node_body(p_ref, nf_ref, spec_ref, wskip_ref, wprod_ref, wpl_ref,
               wread_ref, out1_ref, feats_ref):
    agg = p_ref[0] + p_ref[1]
    spec = spec_ref[...]
    onehot = (spec == lax.broadcasted_iota(jnp.int32, (1, NSPEC), 1)
              ).astype(jnp.float32)
    nf = nf_ref[...]
    skip = jnp.zeros((BN, F), jnp.float32)
    for sp in range(NSPEC):
        skip = skip + onehot[:, sp:sp + 1] * jnp.dot(
            nf, wskip_ref[sp], preferred_element_type=jnp.float32)
    w = jnp.dot(onehot, wprod_ref[...],
                preferred_element_type=jnp.float32)
    pb = jnp.zeros((BN, F), jnp.float32)
    p = agg
    for nu in range(CORR):
        pb = pb + w[:, nu * F:(nu + 1) * F] * p
        p = p * agg
    feats = jnp.dot(pb, wpl_ref[...], preferred_element_type=jnp.float32) + skip
    feats_ref[...] = feats
    out1_ref[...] = jnp.dot(feats, wread_ref[...],
                            preferred_element_type=jnp.float32)


def _node_call(partials, nf, spec2, wskip, wprod2, wpl, wread):
    grid = N // BN
    return pl.pallas_call(
        _node_body,
        grid=(grid,),
        in_specs=[
            pl.BlockSpec((NC, BN, F), lambda i: (0, i, 0)),
            pl.BlockSpec((BN, F), lambda i: (i, 0)),
            pl.BlockSpec((BN, 1), lambda i: (i, 0)),
            pl.BlockSpec((NSPEC, F, F), lambda i: (0, 0, 0)),
            pl.BlockSpec((NSPEC, CORR * F), lambda i: (0, 0)),
            pl.BlockSpec((F, F), lambda i: (0, 0)),
            pl.BlockSpec((F, 1), lambda i: (0, 0)),
        ],
        out_specs=[
            pl.BlockSpec((BN, 1), lambda i: (i, 0)),
            pl.BlockSpec((BN, F), lambda i: (i, 0)),
        ],
        out_shape=[
            jax.ShapeDtypeStruct((N, 1), jnp.float32),
            jax.ShapeDtypeStruct((N, F), jnp.float32),
        ],
    )(partials, nf, spec2, wskip, wprod2, wpl, wread)


def kernel(vectors, node_feats, node_specie, radial_embedding, senders,
           receivers, W_skip, Wr1, br1, Wr2, br2, W_lin, w_prod, W_prodlin,
           W_read):
    pad = E_PAD - E
    snd = jnp.concatenate(
        [senders.astype(jnp.int32), jnp.zeros((pad,), jnp.int32)])
    rcv = jnp.concatenate(
        [receivers.astype(jnp.int32), jnp.full((pad,), N, jnp.int32)])
    recv3 = rcv.reshape(NW, NBATCH, BATCH)
    vec_p = jnp.concatenate([vectors, jnp.ones((pad, 3), jnp.float32)])
    rad_p = jnp.concatenate(
        [radial_embedding, jnp.zeros((pad, NB), jnp.float32)])

    h = _gather()(node_feats, snd)
    m = _edge_call(vec_p, rad_p, h, Wr1, br1.reshape(1, 64), Wr2,
                   br2.reshape(1, SHD), W_lin)
    partials = _scatter()(m, recv3, jnp.zeros((N_PAD, F), jnp.float32))
    out1, feats = _node_call(
        partials[:, :N, :], node_feats,
        node_specie.reshape(N, 1).astype(jnp.int32), W_skip,
        w_prod.reshape(NSPEC, CORR * F), W_prodlin, W_read)
    return out1, feats


# speedup vs baseline: 1.0908x; 1.0908x over previous
import functools

import jax
import jax.numpy as jnp
from jax import lax
from jax.experimental import pallas as pl
from jax.experimental.pallas import tpu as pltpu
from jax.experimental.pallas import tpu_sc as plsc

N = 10000
E = 160000
F = 128
NB = 8
SHD = 9
NSPEC = 10
CORR = 3
AVG = 16.0

NC = 2
NS = 16
NW = NC * NS
EPT = 5120
E_PAD = NW * EPT
BATCH = 128
NBATCH = EPT // BATCH
N_PAD = 10240
RPT = N_PAD // NS

BE = 512
BN = 1000


def _gather_body(nf_hbm, snd_hbm, h_hbm, idx_v, rows0, rows1, gs0, gs1,
                 os0, os1):
    c = lax.axis_index("c")
    s = lax.axis_index("s")
    base = (c * NS + s) * EPT
    pltpu.sync_copy(snd_hbm.at[pl.ds(base, EPT)], idx_v)

    def gat(b, rows, sem):
        off = pl.multiple_of(b * BATCH, BATCH)
        pltpu.async_copy(nf_hbm.at[idx_v.at[pl.ds(off, BATCH)]], rows, sem)

    def out(b, rows, sem):
        off = pl.multiple_of(b * BATCH, BATCH)
        pltpu.async_copy(rows, h_hbm.at[pl.ds(base + off, BATCH)], sem)

    def wait_g(rows, sem):
        pltpu.make_async_copy(nf_hbm.at[idx_v.at[pl.ds(0, BATCH)]], rows,
                              sem).wait()

    def wait_o(rows, sem):
        pltpu.make_async_copy(rows, h_hbm.at[pl.ds(base, BATCH)], sem).wait()

    gat(0, rows0, gs0)

    def body(jj, carry):
        b0 = jj * 2

        @pl.when(jj > 0)
        def _():
            wait_o(rows1, os1)

        gat(b0 + 1, rows1, gs1)
        wait_g(rows0, gs0)
        out(b0, rows0, os0)

        @pl.when(jj < NBATCH // 2 - 1)
        def _():
            wait_o(rows0, os0)
            gat(b0 + 2, rows0, gs0)

        wait_g(rows1, gs1)
        out(b0 + 1, rows1, os1)
        return carry

    lax.fori_loop(0, NBATCH // 2, body, 0)
    wait_o(rows0, os0)
    wait_o(rows1, os1)


@functools.cache
def _gather():
    mesh = plsc.VectorSubcoreMesh(core_axis_name="c", subcore_axis_name="s")
    return pl.kernel(
        _gather_body,
        out_type=jax.ShapeDtypeStruct((E_PAD, F), jnp.float32),
        mesh=mesh,
        scratch_types=[
            pltpu.VMEM((EPT,), jnp.int32),
            pltpu.VMEM((BATCH, F), jnp.float32),
            pltpu.VMEM((BATCH, F), jnp.float32),
            pltpu.SemaphoreType.DMA,
            pltpu.SemaphoreType.DMA,
            pltpu.SemaphoreType.DMA,
            pltpu.SemaphoreType.DMA,
        ],
    )


def _scatter_body(m_hbm, recv3_hbm, zeros_hbm, out_hbm, ridx_v, m0, m1,
                  ls0, ls1, ss0, ss1, acc_sh):
    c = lax.axis_index("c")
    s = lax.axis_index("s")
    wid = c * NS + s
    base = wid * EPT
    row0 = s * RPT
    pltpu.sync_copy(zeros_hbm.at[pl.ds(row0, RPT)], acc_sh.at[pl.ds(row0, RPT)])
    pltpu.sync_copy(recv3_hbm.at[wid], ridx_v)
    plsc.subcore_barrier()

    def load(b, buf, sem):
        off = pl.multiple_of(b * BATCH, BATCH)
        pltpu.async_copy(m_hbm.at[pl.ds(base + off, BATCH)], buf, sem)

    def scat(b, buf, sem):
        pltpu.async_copy(buf, acc_sh.at[ridx_v.at[b]], sem, add=True)

    def wait_l(buf, sem):
        pltpu.make_async_copy(m_hbm.at[pl.ds(base, BATCH)], buf, sem).wait()

    def wait_s(buf, sem):
        pltpu.make_async_copy(buf, acc_sh.at[ridx_v.at[0]], sem).wait()

    load(0, m0, ls0)

    def body(jj, carry):
        b0 = jj * 2

        @pl.when(jj > 0)
        def _():
            wait_s(m1, ss1)

        load(b0 + 1, m1, ls1)
        wait_l(m0, ls0)
        scat(b0, m0, ss0)

        @pl.when(jj < NBATCH // 2 - 1)
        def _():
            wait_s(m0, ss0)
            load(b0 + 2, m0, ls0)

        wait_l(m1, ls1)
        scat(b0 + 1, m1, ss1)
        return carry

    lax.fori_loop(0, NBATCH // 2, body, 0)
    wait_s(m0, ss0)
    wait_s(m1, ss1)
    plsc.subcore_barrier()
    pltpu.sync_copy(acc_sh.at[pl.ds(row0, RPT)], out_hbm.at[c, pl.ds(row0, RPT)])


@functools.cache
def _scatter():
    mesh = plsc.VectorSubcoreMesh(core_axis_name="c", subcore_axis_name="s")
    return pl.kernel(
        _scatter_body,
        out_type=jax.ShapeDtypeStruct((NC, N_PAD, F), jnp.float32),
        mesh=mesh,
        scratch_types=[
            pltpu.VMEM((NBATCH, BATCH), jnp.int32),
            pltpu.VMEM((BATCH, F), jnp.float32),
            pltpu.VMEM((BATCH, F), jnp.float32),
            pltpu.SemaphoreType.DMA,
            pltpu.SemaphoreType.DMA,
            pltpu.SemaphoreType.DMA,
            pltpu.SemaphoreType.DMA,
            pltpu.VMEM_SHARED((N_PAD, F), jnp.float32),
        ],
    )


def _edge_body(vec_ref, rad_ref, h_ref, wr1_ref, br1_ref, wr2_ref, br2_ref,
               wlin_ref, m_ref):
    v = vec_ref[...]
    r = jnp.sqrt(jnp.sum(v * v, axis=1, keepdims=True)) + 1e-8
    u = v / r
    x, y, z = u[:, 0:1], u[:, 1:2], u[:, 2:3]
    yh = jnp.concatenate([
        jnp.ones_like(x), x, y, z,
        x * y, y * z, 3.0 * z * z - 1.0, x * z, x * x - y * y,
    ], axis=1)
    rh = rad_ref[...] @ wr1_ref[...] + br1_ref[...]
    rh = rh * jax.nn.sigmoid(rh)
    rw = rh @ wr2_ref[...] + br2_ref[...]
    cc = yh * rw
    h = h_ref[...]
    x = jnp.concatenate([h * cc[:, lm:lm + 1] for lm in range(SHD)], axis=1)
    acc = jnp.dot(x, wlin_ref[...], preferred_element_type=jnp.float32)
    m_ref[...] = acc * (1.0 / jnp.sqrt(AVG))


def _edge_call(vec_p, rad_p, h, wr1, br1, wr2, br2, wlin3):
    grid = E_PAD // BE
    return pl.pallas_call(
        _edge_body,
        grid=(grid,),
        in_specs=[
            pl.BlockSpec((BE, 3), lambda i: (i, 0)),
            pl.BlockSpec((BE, NB), lambda i: (i, 0)),
            pl.BlockSpec((BE, F), lambda i: (i, 0)),
            pl.BlockSpec((NB, 64), lambda i: (0, 0)),
            pl.BlockSpec((1, 64), lambda i: (0, 0)),
            pl.BlockSpec((64, SHD), lambda i: (0, 0)),
            pl.BlockSpec((1, SHD), lambda i: (0, 0)),
            pl.BlockSpec((SHD * F, F), lambda i: (0, 0)),
        ],
        out_specs=pl.BlockSpec((BE, F), lambda i: (i, 0)),
        out_shape=jax.ShapeDtypeStruct((E_PAD, F), jnp.float32),
    )(vec_p, rad_p, h, wr1, br1, wr2, br2, wlin3)


def _node_body(p_ref, nf_ref, spec_ref, wskip_ref, wprod_ref, wpl_ref,
               wread_ref, out1_ref, feats_ref):
    agg = p_ref[0] + p_ref[1]
    spec = spec_ref[...]
    onehot = (spec == lax.broadcasted_iota(jnp.int32, (1, NSPEC), 1)
              ).astype(jnp.float32)
    nf = nf_ref[...]
    skip = jnp.zeros((BN, F), jnp.float32)
    for sp in range(NSPEC):
        skip = skip + onehot[:, sp:sp + 1] * jnp.dot(
            nf, wskip_ref[sp], preferred_element_type=jnp.float32)
    w = jnp.dot(onehot, wprod_ref[...],
                preferred_element_type=jnp.float32)
    pb = jnp.zeros((BN, F), jnp.float32)
    p = agg
    for nu in range(CORR):
        pb = pb + w[:, nu * F:(nu + 1) * F] * p
        p = p * agg
    feats = jnp.dot(pb, wpl_ref[...], preferred_element_type=jnp.float32) + skip
    feats_ref[...] = feats
    out1_ref[...] = jnp.dot(feats, wread_ref[...],
                            preferred_element_type=jnp.float32)


def _node_call(partials, nf, spec2, wskip, wprod2, wpl, wread):
    grid = N // BN
    return pl.pallas_call(
        _node_body,
        grid=(grid,),
        in_specs=[
            pl.BlockSpec((NC, BN, F), lambda i: (0, i, 0)),
            pl.BlockSpec((BN, F), lambda i: (i, 0)),
            pl.BlockSpec((BN, 1), lambda i: (i, 0)),
            pl.BlockSpec((NSPEC, F, F), lambda i: (0, 0, 0)),
            pl.BlockSpec((NSPEC, CORR * F), lambda i: (0, 0)),
            pl.BlockSpec((F, F), lambda i: (0, 0)),
            pl.BlockSpec((F, 1), lambda i: (0, 0)),
        ],
        out_specs=[
            pl.BlockSpec((BN, 1), lambda i: (i, 0)),
            pl.BlockSpec((BN, F), lambda i: (i, 0)),
        ],
        out_shape=[
            jax.ShapeDtypeStruct((N, 1), jnp.float32),
            jax.ShapeDtypeStruct((N, F), jnp.float32),
        ],
    )(partials, nf, spec2, wskip, wprod2, wpl, wread)


def kernel(vectors, node_feats, node_specie, radial_embedding, senders,
           receivers, W_skip, Wr1, br1, Wr2, br2, W_lin, w_prod, W_prodlin,
           W_read):
    pad = E_PAD - E
    snd = jnp.concatenate(
        [senders.astype(jnp.int32), jnp.zeros((pad,), jnp.int32)])
    rcv = jnp.concatenate(
        [receivers.astype(jnp.int32), jnp.full((pad,), N, jnp.int32)])
    recv3 = rcv.reshape(NW, NBATCH, BATCH)
    vec_p = jnp.concatenate([vectors, jnp.ones((pad, 3), jnp.float32)])
    rad_p = jnp.concatenate(
        [radial_embedding, jnp.zeros((pad, NB), jnp.float32)])

    h = _gather()(node_feats, snd)
    m = _edge_call(vec_p, rad_p, h, Wr1, br1.reshape(1, 64), Wr2,
                   br2.reshape(1, SHD), W_lin)
    partials = _scatter()(m, recv3, jnp.zeros((N_PAD, F), jnp.float32))
    out1, feats = _node_call(
        partials[:, :N, :], node_feats,
        node_specie.reshape(N, 1).astype(jnp.int32), W_skip,
        w_prod.reshape(NSPEC, CORR * F), W_prodlin, W_read)
    return out1, feats


# speedup vs baseline: 1.8684x; 1.7129x over previous
import functools

import jax
import jax.numpy as jnp
from jax import lax
from jax.experimental import pallas as pl
from jax.experimental.pallas import tpu as pltpu
from jax.experimental.pallas import tpu_sc as plsc

N = 10000
E = 160000
F = 128
NB = 8
SHD = 9
NSPEC = 10
CORR = 3
AVG = 16.0

NC = 2
NS = 16
NW = NC * NS
EPT = 5120
E_PAD = NW * EPT
BATCH = 128
NBATCH = EPT // BATCH
N_PAD = 10240
RPT = N_PAD // NS

BE = 2048
BN = 1000


def _gather_body(nf_hbm, snd_hbm, h_hbm, idx_v, rows0, rows1, gs0, gs1,
                 os0, os1):
    c = lax.axis_index("c")
    s = lax.axis_index("s")
    base = (c * NS + s) * EPT
    pltpu.sync_copy(snd_hbm.at[pl.ds(base, EPT)], idx_v)

    def gat(b, rows, sem):
        off = pl.multiple_of(b * BATCH, BATCH)
        pltpu.async_copy(nf_hbm.at[idx_v.at[pl.ds(off, BATCH)]], rows, sem)

    def out(b, rows, sem):
        off = pl.multiple_of(b * BATCH, BATCH)
        pltpu.async_copy(rows, h_hbm.at[pl.ds(base + off, BATCH)], sem)

    def wait_g(rows, sem):
        pltpu.make_async_copy(nf_hbm.at[idx_v.at[pl.ds(0, BATCH)]], rows,
                              sem).wait()

    def wait_o(rows, sem):
        pltpu.make_async_copy(rows, h_hbm.at[pl.ds(base, BATCH)], sem).wait()

    gat(0, rows0, gs0)

    def body(jj, carry):
        b0 = jj * 2

        @pl.when(jj > 0)
        def _():
            wait_o(rows1, os1)

        gat(b0 + 1, rows1, gs1)
        wait_g(rows0, gs0)
        out(b0, rows0, os0)

        @pl.when(jj < NBATCH // 2 - 1)
        def _():
            wait_o(rows0, os0)
            gat(b0 + 2, rows0, gs0)

        wait_g(rows1, gs1)
        out(b0 + 1, rows1, os1)
        return carry

    lax.fori_loop(0, NBATCH // 2, body, 0)
    wait_o(rows0, os0)
    wait_o(rows1, os1)


@functools.cache
def _gather():
    mesh = plsc.VectorSubcoreMesh(core_axis_name="c", subcore_axis_name="s")
    return pl.kernel(
        _gather_body,
        out_type=jax.ShapeDtypeStruct((E_PAD, F), jnp.float32),
        mesh=mesh,
        scratch_types=[
            pltpu.VMEM((EPT,), jnp.int32),
            pltpu.VMEM((BATCH, F), jnp.float32),
            pltpu.VMEM((BATCH, F), jnp.float32),
            pltpu.SemaphoreType.DMA,
            pltpu.SemaphoreType.DMA,
            pltpu.SemaphoreType.DMA,
            pltpu.SemaphoreType.DMA,
        ],
    )


def _scatter_body(m_hbm, recv3_hbm, zeros_hbm, out_hbm, ridx_v, m0, m1,
                  ls0, ls1, ss0, ss1, acc_sh):
    c = lax.axis_index("c")
    s = lax.axis_index("s")
    wid = c * NS + s
    base = wid * EPT
    row0 = s * RPT
    pltpu.sync_copy(zeros_hbm.at[pl.ds(row0, RPT)], acc_sh.at[pl.ds(row0, RPT)])
    pltpu.sync_copy(recv3_hbm.at[wid], ridx_v)
    plsc.subcore_barrier()

    def load(b, buf, sem):
        off = pl.multiple_of(b * BATCH, BATCH)
        pltpu.async_copy(m_hbm.at[pl.ds(base + off, BATCH)], buf, sem)

    def scat(b, buf, sem):
        pltpu.async_copy(buf, acc_sh.at[ridx_v.at[b]], sem, add=True)

    def wait_l(buf, sem):
        pltpu.make_async_copy(m_hbm.at[pl.ds(base, BATCH)], buf, sem).wait()

    def wait_s(buf, sem):
        pltpu.make_async_copy(buf, acc_sh.at[ridx_v.at[0]], sem).wait()

    load(0, m0, ls0)

    def body(jj, carry):
        b0 = jj * 2

        @pl.when(jj > 0)
        def _():
            wait_s(m1, ss1)

        load(b0 + 1, m1, ls1)
        wait_l(m0, ls0)
        scat(b0, m0, ss0)

        @pl.when(jj < NBATCH // 2 - 1)
        def _():
            wait_s(m0, ss0)
            load(b0 + 2, m0, ls0)

        wait_l(m1, ls1)
        scat(b0 + 1, m1, ss1)
        return carry

    lax.fori_loop(0, NBATCH // 2, body, 0)
    wait_s(m0, ss0)
    wait_s(m1, ss1)
    plsc.subcore_barrier()
    pltpu.sync_copy(acc_sh.at[pl.ds(row0, RPT)], out_hbm.at[c, pl.ds(row0, RPT)])


@functools.cache
def _scatter():
    mesh = plsc.VectorSubcoreMesh(core_axis_name="c", subcore_axis_name="s")
    return pl.kernel(
        _scatter_body,
        out_type=jax.ShapeDtypeStruct((NC, N_PAD, F), jnp.float32),
        mesh=mesh,
        scratch_types=[
            pltpu.VMEM((NBATCH, BATCH), jnp.int32),
            pltpu.VMEM((BATCH, F), jnp.float32),
            pltpu.VMEM((BATCH, F), jnp.float32),
            pltpu.SemaphoreType.DMA,
            pltpu.SemaphoreType.DMA,
            pltpu.SemaphoreType.DMA,
            pltpu.SemaphoreType.DMA,
            pltpu.VMEM_SHARED((N_PAD, F), jnp.float32),
        ],
    )


def _edge_body(vect_ref, radt_ref, h_ref, wr1t_ref, br1_ref, wr2t_ref,
               br2_ref, exp_ref, wlin_ref, m_ref):
    vt = vect_ref[...]
    r = jnp.sqrt(jnp.sum(vt * vt, axis=0, keepdims=True)) + 1e-8
    ut = vt / r
    x, y, z = ut[0:1, :], ut[1:2, :], ut[2:3, :]
    yht = jnp.concatenate([
        jnp.ones_like(x), x, y, z,
        x * y, y * z, 3.0 * z * z - 1.0, x * z, x * x - y * y,
    ], axis=0)
    rht = jnp.dot(wr1t_ref[...], radt_ref[...],
                  preferred_element_type=jnp.float32) + br1_ref[...]
    rht = rht * jax.nn.sigmoid(rht)
    rwt = jnp.dot(wr2t_ref[...], rht,
                  preferred_element_type=jnp.float32) + br2_ref[...]
    cct = yht * rwt
    ccb = lax.dot_general(cct, exp_ref[...], (((0,), (0,)), ((), ())),
                          preferred_element_type=jnp.float32)
    h = h_ref[...]
    ht = jnp.concatenate([h] * SHD, axis=1)
    acc = jnp.dot(ccb * ht, wlin_ref[...],
                  preferred_element_type=jnp.float32)
    m_ref[...] = acc * (1.0 / jnp.sqrt(AVG))


def _edge_call(vec_p, rad_p, h, wr1, br1, wr2, br2, wlin):
    grid = E_PAD // BE
    exp9 = jnp.kron(jnp.eye(SHD, dtype=jnp.float32),
                    jnp.ones((1, F), jnp.float32))
    return pl.pallas_call(
        _edge_body,
        grid=(grid,),
        in_specs=[
            pl.BlockSpec((3, BE), lambda i: (0, i)),
            pl.BlockSpec((NB, BE), lambda i: (0, i)),
            pl.BlockSpec((BE, F), lambda i: (i, 0)),
            pl.BlockSpec((64, NB), lambda i: (0, 0)),
            pl.BlockSpec((64, 1), lambda i: (0, 0)),
            pl.BlockSpec((SHD, 64), lambda i: (0, 0)),
            pl.BlockSpec((SHD, 1), lambda i: (0, 0)),
            pl.BlockSpec((SHD, SHD * F), lambda i: (0, 0)),
            pl.BlockSpec((SHD * F, F), lambda i: (0, 0)),
        ],
        out_specs=pl.BlockSpec((BE, F), lambda i: (i, 0)),
        out_shape=jax.ShapeDtypeStruct((E_PAD, F), jnp.float32),
    )(vec_p.T, rad_p.T, h, wr1.T, br1.reshape(64, 1), wr2.T,
      br2.reshape(SHD, 1), exp9, wlin)


def _node_body(p_ref, nf_ref, spec_ref, wskip_ref, wprod_ref, wpl_ref,
               wread_ref, out1_ref, feats_ref):
    agg = p_ref[0] + p_ref[1]
    spec = spec_ref[...]
    onehot = (spec == lax.broadcasted_iota(jnp.int32, (1, NSPEC), 1)
              ).astype(jnp.float32)
    nf = nf_ref[...]
    skip = jnp.zeros((BN, F), jnp.float32)
    for sp in range(NSPEC):
        skip = skip + onehot[:, sp:sp + 1] * jnp.dot(
            nf, wskip_ref[sp], preferred_element_type=jnp.float32)
    w = jnp.dot(onehot, wprod_ref[...],
                preferred_element_type=jnp.float32)
    pb = jnp.zeros((BN, F), jnp.float32)
    p = agg
    for nu in range(CORR):
        pb = pb + w[:, nu * F:(nu + 1) * F] * p
        p = p * agg
    feats = jnp.dot(pb, wpl_ref[...], preferred_element_type=jnp.float32) + skip
    feats_ref[...] = feats
    out1_ref[...] = jnp.dot(feats, wread_ref[...],
                            preferred_element_type=jnp.float32)


def _node_call(partials, nf, spec2, wskip, wprod2, wpl, wread):
    grid = N // BN
    return pl.pallas_call(
        _node_body,
        grid=(grid,),
        in_specs=[
            pl.BlockSpec((NC, BN, F), lambda i: (0, i, 0)),
            pl.BlockSpec((BN, F), lambda i: (i, 0)),
            pl.BlockSpec((BN, 1), lambda i: (i, 0)),
            pl.BlockSpec((NSPEC, F, F), lambda i: (0, 0, 0)),
            pl.BlockSpec((NSPEC, CORR * F), lambda i: (0, 0)),
            pl.BlockSpec((F, F), lambda i: (0, 0)),
            pl.BlockSpec((F, 1), lambda i: (0, 0)),
        ],
        out_specs=[
            pl.BlockSpec((BN, 1), lambda i: (i, 0)),
            pl.BlockSpec((BN, F), lambda i: (i, 0)),
        ],
        out_shape=[
            jax.ShapeDtypeStruct((N, 1), jnp.float32),
            jax.ShapeDtypeStruct((N, F), jnp.float32),
        ],
    )(partials, nf, spec2, wskip, wprod2, wpl, wread)


def kernel(vectors, node_feats, node_specie, radial_embedding, senders,
           receivers, W_skip, Wr1, br1, Wr2, br2, W_lin, w_prod, W_prodlin,
           W_read):
    pad = E_PAD - E
    snd = jnp.concatenate(
        [senders.astype(jnp.int32), jnp.zeros((pad,), jnp.int32)])
    rcv = jnp.concatenate(
        [receivers.astype(jnp.int32), jnp.full((pad,), N, jnp.int32)])
    recv3 = rcv.reshape(NW, NBATCH, BATCH)
    vec_p = jnp.concatenate([vectors, jnp.ones((pad, 3), jnp.float32)])
    rad_p = jnp.concatenate(
        [radial_embedding, jnp.zeros((pad, NB), jnp.float32)])

    h = _gather()(node_feats, snd)
    m = _edge_call(vec_p, rad_p, h, Wr1, br1, Wr2, br2, W_lin)
    partials = _scatter()(m, recv3, jnp.zeros((N_PAD, F), jnp.float32))
    out1, feats = _node_call(
        partials[:, :N, :], node_feats,
        node_specie.reshape(N, 1).astype(jnp.int32), W_skip,
        w_prod.reshape(NSPEC, CORR * F), W_prodlin, W_read)
    return out1, feats


# speedup vs baseline: 1.8687x; 1.0002x over previous
import functools

import jax
import jax.numpy as jnp
from jax import lax
from jax.experimental import pallas as pl
from jax.experimental.pallas import tpu as pltpu
from jax.experimental.pallas import tpu_sc as plsc

N = 10000
E = 160000
F = 128
NB = 8
SHD = 9
NSPEC = 10
CORR = 3
AVG = 16.0

NC = 2
NS = 16
NW = NC * NS
EPT = 5120
E_PAD = NW * EPT
BATCH = 128
NBATCH = EPT // BATCH
N_PAD = 10240
RPT = N_PAD // NS

BE = 2048
BN = 1000


def _gather_body(nf_hbm, snd_hbm, h_hbm, idx_v, rows0, rows1, gs0, gs1,
                 os0, os1):
    c = lax.axis_index("c")
    s = lax.axis_index("s")
    base = (c * NS + s) * EPT
    pltpu.sync_copy(snd_hbm.at[pl.ds(base, EPT)], idx_v)

    def gat(b, rows, sem):
        off = pl.multiple_of(b * BATCH, BATCH)
        pltpu.async_copy(nf_hbm.at[idx_v.at[pl.ds(off, BATCH)]], rows, sem)

    def out(b, rows, sem):
        off = pl.multiple_of(b * BATCH, BATCH)
        pltpu.async_copy(rows, h_hbm.at[pl.ds(base + off, BATCH)], sem)

    def wait_g(rows, sem):
        pltpu.make_async_copy(nf_hbm.at[idx_v.at[pl.ds(0, BATCH)]], rows,
                              sem).wait()

    def wait_o(rows, sem):
        pltpu.make_async_copy(rows, h_hbm.at[pl.ds(base, BATCH)], sem).wait()

    gat(0, rows0, gs0)

    def body(jj, carry):
        b0 = jj * 2

        @pl.when(jj > 0)
        def _():
            wait_o(rows1, os1)

        gat(b0 + 1, rows1, gs1)
        wait_g(rows0, gs0)
        out(b0, rows0, os0)

        @pl.when(jj < NBATCH // 2 - 1)
        def _():
            wait_o(rows0, os0)
            gat(b0 + 2, rows0, gs0)

        wait_g(rows1, gs1)
        out(b0 + 1, rows1, os1)
        return carry

    lax.fori_loop(0, NBATCH // 2, body, 0)
    wait_o(rows0, os0)
    wait_o(rows1, os1)


@functools.cache
def _gather():
    mesh = plsc.VectorSubcoreMesh(core_axis_name="c", subcore_axis_name="s")
    return pl.kernel(
        _gather_body,
        out_type=jax.ShapeDtypeStruct((E_PAD, F), jnp.float32),
        mesh=mesh,
        scratch_types=[
            pltpu.VMEM((EPT,), jnp.int32),
            pltpu.VMEM((BATCH, F), jnp.float32),
            pltpu.VMEM((BATCH, F), jnp.float32),
            pltpu.SemaphoreType.DMA,
            pltpu.SemaphoreType.DMA,
            pltpu.SemaphoreType.DMA,
            pltpu.SemaphoreType.DMA,
        ],
    )


def _scatter_body(m_hbm, recv3_hbm, zeros_hbm, out_hbm, ridx_v, m0, m1,
                  ls0, ls1, ss0, ss1, acc_sh):
    c = lax.axis_index("c")
    s = lax.axis_index("s")
    wid = c * NS + s
    base = wid * EPT
    row0 = s * RPT
    pltpu.sync_copy(zeros_hbm.at[pl.ds(row0, RPT)], acc_sh.at[pl.ds(row0, RPT)])
    pltpu.sync_copy(recv3_hbm.at[wid], ridx_v)
    plsc.subcore_barrier()

    def load(b, buf, sem):
        off = pl.multiple_of(b * BATCH, BATCH)
        pltpu.async_copy(m_hbm.at[pl.ds(base + off, BATCH)], buf, sem)

    def scat(b, buf, sem):
        pltpu.async_copy(buf, acc_sh.at[ridx_v.at[b]], sem, add=True)

    def wait_l(buf, sem):
        pltpu.make_async_copy(m_hbm.at[pl.ds(base, BATCH)], buf, sem).wait()

    def wait_s(buf, sem):
        pltpu.make_async_copy(buf, acc_sh.at[ridx_v.at[0]], sem).wait()

    load(0, m0, ls0)

    def body(jj, carry):
        b0 = jj * 2

        @pl.when(jj > 0)
        def _():
            wait_s(m1, ss1)

        load(b0 + 1, m1, ls1)
        wait_l(m0, ls0)
        scat(b0, m0, ss0)

        @pl.when(jj < NBATCH // 2 - 1)
        def _():
            wait_s(m0, ss0)
            load(b0 + 2, m0, ls0)

        wait_l(m1, ls1)
        scat(b0 + 1, m1, ss1)
        return carry

    lax.fori_loop(0, NBATCH // 2, body, 0)
    wait_s(m0, ss0)
    wait_s(m1, ss1)
    plsc.subcore_barrier()
    pltpu.sync_copy(acc_sh.at[pl.ds(row0, RPT)], out_hbm.at[c, pl.ds(row0, RPT)])


@functools.cache
def _scatter():
    mesh = plsc.VectorSubcoreMesh(core_axis_name="c", subcore_axis_name="s")
    return pl.kernel(
        _scatter_body,
        out_type=jax.ShapeDtypeStruct((NC, N_PAD, F), jnp.float32),
        mesh=mesh,
        scratch_types=[
            pltpu.VMEM((NBATCH, BATCH), jnp.int32),
            pltpu.VMEM((BATCH, F), jnp.float32),
            pltpu.VMEM((BATCH, F), jnp.float32),
            pltpu.SemaphoreType.DMA,
            pltpu.SemaphoreType.DMA,
            pltpu.SemaphoreType.DMA,
            pltpu.SemaphoreType.DMA,
            pltpu.VMEM_SHARED((N_PAD, F), jnp.float32),
        ],
    )


def _edge_body(vect_ref, radt_ref, h_ref, wr1t_ref, br1_ref, wr2t_ref,
               br2_ref, exp_ref, wlin_ref, m_ref):
    vt = vect_ref[...]
    r = jnp.sqrt(jnp.sum(vt * vt, axis=0, keepdims=True)) + 1e-8
    ut = vt / r
    x, y, z = ut[0:1, :], ut[1:2, :], ut[2:3, :]
    yht = jnp.concatenate([
        jnp.ones_like(x), x, y, z,
        x * y, y * z, 3.0 * z * z - 1.0, x * z, x * x - y * y,
    ], axis=0)
    rht = jnp.dot(wr1t_ref[...], radt_ref[...],
                  preferred_element_type=jnp.float32) + br1_ref[...]
    rht = rht * jax.nn.sigmoid(rht)
    rwt = jnp.dot(wr2t_ref[...], rht,
                  preferred_element_type=jnp.float32) + br2_ref[...]
    cct = yht * rwt
    cc = cct.astype(jnp.bfloat16).T
    h = h_ref[...].astype(jnp.bfloat16)
    x = jnp.concatenate([h * cc[:, lm:lm + 1] for lm in range(SHD)],
                        axis=1)
    acc = jnp.dot(x, wlin_ref[...], preferred_element_type=jnp.float32)
    m_ref[...] = acc * (1.0 / jnp.sqrt(AVG))


def _edge_call(vec_p, rad_p, h, wr1, br1, wr2, br2, wlin):
    grid = E_PAD // BE
    exp9 = jnp.kron(jnp.eye(SHD, dtype=jnp.bfloat16),
                    jnp.ones((1, F), jnp.bfloat16))
    return pl.pallas_call(
        _edge_body,
        grid=(grid,),
        in_specs=[
            pl.BlockSpec((3, BE), lambda i: (0, i)),
            pl.BlockSpec((NB, BE), lambda i: (0, i)),
            pl.BlockSpec((BE, F), lambda i: (i, 0)),
            pl.BlockSpec((64, NB), lambda i: (0, 0)),
            pl.BlockSpec((64, 1), lambda i: (0, 0)),
            pl.BlockSpec((SHD, 64), lambda i: (0, 0)),
            pl.BlockSpec((SHD, 1), lambda i: (0, 0)),
            pl.BlockSpec((SHD, SHD * F), lambda i: (0, 0)),
            pl.BlockSpec((SHD * F, F), lambda i: (0, 0)),
        ],
        out_specs=pl.BlockSpec((BE, F), lambda i: (i, 0)),
        out_shape=jax.ShapeDtypeStruct((E_PAD, F), jnp.float32),
    )(vec_p.T, rad_p.T, h, wr1.T, br1.reshape(64, 1), wr2.T,
      br2.reshape(SHD, 1), exp9, wlin.astype(jnp.bfloat16))


def _node_body(p_ref, nf_ref, spec_ref, wskip_ref, wprod_ref, wpl_ref,
               wread_ref, out1_ref, feats_ref):
    agg = p_ref[0] + p_ref[1]
    spec = spec_ref[...]
    onehot = (spec == lax.broadcasted_iota(jnp.int32, (1, NSPEC), 1)
              ).astype(jnp.float32)
    nf = nf_ref[...]
    skip = jnp.zeros((BN, F), jnp.float32)
    for sp in range(NSPEC):
        skip = skip + onehot[:, sp:sp + 1] * jnp.dot(
            nf, wskip_ref[sp], preferred_element_type=jnp.float32)
    w = jnp.dot(onehot, wprod_ref[...],
                preferred_element_type=jnp.float32)
    pb = jnp.zeros((BN, F), jnp.float32)
    p = agg
    for nu in range(CORR):
        pb = pb + w[:, nu * F:(nu + 1) * F] * p
        p = p * agg
    feats = jnp.dot(pb, wpl_ref[...], preferred_element_type=jnp.float32) + skip
    feats_ref[...] = feats
    out1_ref[...] = jnp.dot(feats, wread_ref[...],
                            preferred_element_type=jnp.float32)


def _node_call(partials, nf, spec2, wskip, wprod2, wpl, wread):
    grid = N // BN
    return pl.pallas_call(
        _node_body,
        grid=(grid,),
        in_specs=[
            pl.BlockSpec((NC, BN, F), lambda i: (0, i, 0)),
            pl.BlockSpec((BN, F), lambda i: (i, 0)),
            pl.BlockSpec((BN, 1), lambda i: (i, 0)),
            pl.BlockSpec((NSPEC, F, F), lambda i: (0, 0, 0)),
            pl.BlockSpec((NSPEC, CORR * F), lambda i: (0, 0)),
            pl.BlockSpec((F, F), lambda i: (0, 0)),
            pl.BlockSpec((F, 1), lambda i: (0, 0)),
        ],
        out_specs=[
            pl.BlockSpec((BN, 1), lambda i: (i, 0)),
            pl.BlockSpec((BN, F), lambda i: (i, 0)),
        ],
        out_shape=[
            jax.ShapeDtypeStruct((N, 1), jnp.float32),
            jax.ShapeDtypeStruct((N, F), jnp.float32),
        ],
    )(partials, nf, spec2, wskip, wprod2, wpl, wread)


def kernel(vectors, node_feats, node_specie, radial_embedding, senders,
           receivers, W_skip, Wr1, br1, Wr2, br2, W_lin, w_prod, W_prodlin,
           W_read):
    pad = E_PAD - E
    snd = jnp.concatenate(
        [senders.astype(jnp.int32), jnp.zeros((pad,), jnp.int32)])
    rcv = jnp.concatenate(
        [receivers.astype(jnp.int32), jnp.full((pad,), N, jnp.int32)])
    recv3 = rcv.reshape(NW, NBATCH, BATCH)
    vec_p = jnp.concatenate([vectors, jnp.ones((pad, 3), jnp.float32)])
    rad_p = jnp.concatenate(
        [radial_embedding, jnp.zeros((pad, NB), jnp.float32)])

    h = _gather()(node_feats, snd)
    m = _edge_call(vec_p, rad_p, h, Wr1, br1, Wr2, br2, W_lin)
    partials = _scatter()(m, recv3, jnp.zeros((N_PAD, F), jnp.float32))
    out1, feats = _node_call(
        partials[:, :N, :], node_feats,
        node_specie.reshape(N, 1).astype(jnp.int32), W_skip,
        w_prod.reshape(NSPEC, CORR * F), W_prodlin, W_read)
    return out1, feats


# speedup vs baseline: 1.9744x; 1.0566x over previous
import functools

import jax
import jax.numpy as jnp
from jax import lax
from jax.experimental import pallas as pl
from jax.experimental.pallas import tpu as pltpu
from jax.experimental.pallas import tpu_sc as plsc

N = 10000
E = 160000
F = 128
NB = 8
SHD = 9
NSPEC = 10
CORR = 3
AVG = 16.0

NC = 2
NS = 16
NW = NC * NS
EPT = 5120
E_PAD = NW * EPT
BATCH = 128
NBATCH = EPT // BATCH
N_PAD = 10240
RPT = N_PAD // NS

BE = 2048
BN = 1000
NCHUNK = 2
CH = E_PAD // NCHUNK
EPT_C = CH // NW
NBATCH_C = EPT_C // BATCH


def _gather_body(nf_hbm, snd_hbm, h_hbm, idx_v, rows0, rows1, gs0, gs1,
                 os0, os1):
    c = lax.axis_index("c")
    s = lax.axis_index("s")
    base = (c * NS + s) * EPT_C
    pltpu.sync_copy(snd_hbm.at[pl.ds(base, EPT_C)], idx_v)

    def gat(b, rows, sem):
        off = pl.multiple_of(b * BATCH, BATCH)
        pltpu.async_copy(nf_hbm.at[idx_v.at[pl.ds(off, BATCH)]], rows, sem)

    def out(b, rows, sem):
        off = pl.multiple_of(b * BATCH, BATCH)
        pltpu.async_copy(rows, h_hbm.at[pl.ds(base + off, BATCH)], sem)

    def wait_g(rows, sem):
        pltpu.make_async_copy(nf_hbm.at[idx_v.at[pl.ds(0, BATCH)]], rows,
                              sem).wait()

    def wait_o(rows, sem):
        pltpu.make_async_copy(rows, h_hbm.at[pl.ds(base, BATCH)], sem).wait()

    gat(0, rows0, gs0)

    def body(jj, carry):
        b0 = jj * 2

        @pl.when(jj > 0)
        def _():
            wait_o(rows1, os1)

        gat(b0 + 1, rows1, gs1)
        wait_g(rows0, gs0)
        out(b0, rows0, os0)

        @pl.when(jj < NBATCH_C // 2 - 1)
        def _():
            wait_o(rows0, os0)
            gat(b0 + 2, rows0, gs0)

        wait_g(rows1, gs1)
        out(b0 + 1, rows1, os1)
        return carry

    lax.fori_loop(0, NBATCH_C // 2, body, 0)
    wait_o(rows0, os0)
    wait_o(rows1, os1)


@functools.cache
def _gather():
    mesh = plsc.VectorSubcoreMesh(core_axis_name="c", subcore_axis_name="s")
    return pl.kernel(
        _gather_body,
        out_type=jax.ShapeDtypeStruct((CH, F), jnp.float32),
        mesh=mesh,
        scratch_types=[
            pltpu.VMEM((EPT_C,), jnp.int32),
            pltpu.VMEM((BATCH, F), jnp.float32),
            pltpu.VMEM((BATCH, F), jnp.float32),
            pltpu.SemaphoreType.DMA,
            pltpu.SemaphoreType.DMA,
            pltpu.SemaphoreType.DMA,
            pltpu.SemaphoreType.DMA,
        ],
    )


def _scatter_body(m_hbm, recv3_hbm, zeros_hbm, out_hbm, ridx_v, m0, m1,
                  ls0, ls1, ss0, ss1, acc_sh):
    c = lax.axis_index("c")
    s = lax.axis_index("s")
    wid = c * NS + s
    base = wid * EPT_C
    row0 = s * RPT
    pltpu.sync_copy(zeros_hbm.at[pl.ds(row0, RPT)], acc_sh.at[pl.ds(row0, RPT)])
    pltpu.sync_copy(recv3_hbm.at[wid], ridx_v)
    plsc.subcore_barrier()

    def load(b, buf, sem):
        off = pl.multiple_of(b * BATCH, BATCH)
        pltpu.async_copy(m_hbm.at[pl.ds(base + off, BATCH)], buf, sem)

    def scat(b, buf, sem):
        pltpu.async_copy(buf, acc_sh.at[ridx_v.at[b]], sem, add=True)

    def wait_l(buf, sem):
        pltpu.make_async_copy(m_hbm.at[pl.ds(base, BATCH)], buf, sem).wait()

    def wait_s(buf, sem):
        pltpu.make_async_copy(buf, acc_sh.at[ridx_v.at[0]], sem).wait()

    load(0, m0, ls0)

    def body(jj, carry):
        b0 = jj * 2

        @pl.when(jj > 0)
        def _():
            wait_s(m1, ss1)

        load(b0 + 1, m1, ls1)
        wait_l(m0, ls0)
        scat(b0, m0, ss0)

        @pl.when(jj < NBATCH_C // 2 - 1)
        def _():
            wait_s(m0, ss0)
            load(b0 + 2, m0, ls0)

        wait_l(m1, ls1)
        scat(b0 + 1, m1, ss1)
        return carry

    lax.fori_loop(0, NBATCH_C // 2, body, 0)
    wait_s(m0, ss0)
    wait_s(m1, ss1)
    plsc.subcore_barrier()
    pltpu.sync_copy(acc_sh.at[pl.ds(row0, RPT)], out_hbm.at[c, pl.ds(row0, RPT)])


@functools.cache
def _scatter():
    mesh = plsc.VectorSubcoreMesh(core_axis_name="c", subcore_axis_name="s")
    return pl.kernel(
        _scatter_body,
        out_type=jax.ShapeDtypeStruct((NC, N_PAD, F), jnp.float32),
        mesh=mesh,
        scratch_types=[
            pltpu.VMEM((NBATCH_C, BATCH), jnp.int32),
            pltpu.VMEM((BATCH, F), jnp.float32),
            pltpu.VMEM((BATCH, F), jnp.float32),
            pltpu.SemaphoreType.DMA,
            pltpu.SemaphoreType.DMA,
            pltpu.SemaphoreType.DMA,
            pltpu.SemaphoreType.DMA,
            pltpu.VMEM_SHARED((N_PAD, F), jnp.float32),
        ],
    )


def _edge_body(vect_ref, radt_ref, h_ref, wr1t_ref, br1_ref, wr2t_ref,
               br2_ref, exp_ref, wlin_ref, m_ref):
    vt = vect_ref[...]
    r = jnp.sqrt(jnp.sum(vt * vt, axis=0, keepdims=True)) + 1e-8
    ut = vt / r
    x, y, z = ut[0:1, :], ut[1:2, :], ut[2:3, :]
    yht = jnp.concatenate([
        jnp.ones_like(x), x, y, z,
        x * y, y * z, 3.0 * z * z - 1.0, x * z, x * x - y * y,
    ], axis=0)
    rht = jnp.dot(wr1t_ref[...], radt_ref[...],
                  preferred_element_type=jnp.float32) + br1_ref[...]
    rht = rht * jax.nn.sigmoid(rht)
    rwt = jnp.dot(wr2t_ref[...], rht,
                  preferred_element_type=jnp.float32) + br2_ref[...]
    cct = yht * rwt
    cc = cct.astype(jnp.bfloat16).T
    h = h_ref[...].astype(jnp.bfloat16)
    x = jnp.concatenate([h * cc[:, lm:lm + 1] for lm in range(SHD)],
                        axis=1)
    acc = jnp.dot(x, wlin_ref[...], preferred_element_type=jnp.float32)
    m_ref[...] = acc * (1.0 / jnp.sqrt(AVG))


def _edge_call(vec_p, rad_p, h, wr1, br1, wr2, br2, wlin):
    grid = CH // BE
    exp9 = jnp.kron(jnp.eye(SHD, dtype=jnp.bfloat16),
                    jnp.ones((1, F), jnp.bfloat16))
    return pl.pallas_call(
        _edge_body,
        grid=(grid,),
        in_specs=[
            pl.BlockSpec((3, BE), lambda i: (0, i)),
            pl.BlockSpec((NB, BE), lambda i: (0, i)),
            pl.BlockSpec((BE, F), lambda i: (i, 0)),
            pl.BlockSpec((64, NB), lambda i: (0, 0)),
            pl.BlockSpec((64, 1), lambda i: (0, 0)),
            pl.BlockSpec((SHD, 64), lambda i: (0, 0)),
            pl.BlockSpec((SHD, 1), lambda i: (0, 0)),
            pl.BlockSpec((SHD, SHD * F), lambda i: (0, 0)),
            pl.BlockSpec((SHD * F, F), lambda i: (0, 0)),
        ],
        out_specs=pl.BlockSpec((BE, F), lambda i: (i, 0)),
        out_shape=jax.ShapeDtypeStruct((CH, F), jnp.float32),
    )(vec_p.T, rad_p.T, h, wr1.T, br1.reshape(64, 1), wr2.T,
      br2.reshape(SHD, 1), exp9, wlin.astype(jnp.bfloat16))


def _node_body(p_ref, nf_ref, spec_ref, wskip_ref, wprod_ref, wpl_ref,
               wread_ref, out1_ref, feats_ref):
    agg = jnp.sum(p_ref[...], axis=0)
    spec = spec_ref[...]
    onehot = (spec == lax.broadcasted_iota(jnp.int32, (1, NSPEC), 1)
              ).astype(jnp.float32)
    nf = nf_ref[...]
    skip = jnp.zeros((BN, F), jnp.float32)
    for sp in range(NSPEC):
        skip = skip + onehot[:, sp:sp + 1] * jnp.dot(
            nf, wskip_ref[sp], preferred_element_type=jnp.float32)
    w = jnp.dot(onehot, wprod_ref[...],
                preferred_element_type=jnp.float32)
    pb = jnp.zeros((BN, F), jnp.float32)
    p = agg
    for nu in range(CORR):
        pb = pb + w[:, nu * F:(nu + 1) * F] * p
        p = p * agg
    feats = jnp.dot(pb, wpl_ref[...], preferred_element_type=jnp.float32) + skip
    feats_ref[...] = feats
    out1_ref[...] = jnp.dot(feats, wread_ref[...],
                            preferred_element_type=jnp.float32)


def _node_call(partials, nf, spec2, wskip, wprod2, wpl, wread):
    grid = N // BN
    return pl.pallas_call(
        _node_body,
        grid=(grid,),
        in_specs=[
            pl.BlockSpec((NCHUNK * NC, BN, F), lambda i: (0, i, 0)),
            pl.BlockSpec((BN, F), lambda i: (i, 0)),
            pl.BlockSpec((BN, 1), lambda i: (i, 0)),
            pl.BlockSpec((NSPEC, F, F), lambda i: (0, 0, 0)),
            pl.BlockSpec((NSPEC, CORR * F), lambda i: (0, 0)),
            pl.BlockSpec((F, F), lambda i: (0, 0)),
            pl.BlockSpec((F, 1), lambda i: (0, 0)),
        ],
        out_specs=[
            pl.BlockSpec((BN, 1), lambda i: (i, 0)),
            pl.BlockSpec((BN, F), lambda i: (i, 0)),
        ],
        out_shape=[
            jax.ShapeDtypeStruct((N, 1), jnp.float32),
            jax.ShapeDtypeStruct((N, F), jnp.float32),
        ],
    )(partials, nf, spec2, wskip, wprod2, wpl, wread)


def kernel(vectors, node_feats, node_specie, radial_embedding, senders,
           receivers, W_skip, Wr1, br1, Wr2, br2, W_lin, w_prod, W_prodlin,
           W_read):
    pad = E_PAD - E
    snd = jnp.concatenate(
        [senders.astype(jnp.int32), jnp.zeros((pad,), jnp.int32)])
    rcv = jnp.concatenate(
        [receivers.astype(jnp.int32), jnp.full((pad,), N, jnp.int32)])
    vec_p = jnp.concatenate([vectors, jnp.ones((pad, 3), jnp.float32)])
    rad_p = jnp.concatenate(
        [radial_embedding, jnp.zeros((pad, NB), jnp.float32)])
    zeros = jnp.zeros((N_PAD, F), jnp.float32)

    partials = []
    for k in range(NCHUNK):
        sl = slice(k * CH, (k + 1) * CH)
        h = _gather()(node_feats, snd[sl])
        m = _edge_call(vec_p[sl], rad_p[sl], h, Wr1, br1, Wr2, br2, W_lin)
        recv3 = rcv[sl].reshape(NW, NBATCH_C, BATCH)
        partials.append(_scatter()(m, recv3, zeros))
    out1, feats = _node_call(
        jnp.concatenate(partials, axis=0)[:, :N, :], node_feats,
        node_specie.reshape(N, 1).astype(jnp.int32), W_skip,
        w_prod.reshape(NSPEC, CORR * F), W_prodlin, W_read)
    return out1, feats


# speedup vs baseline: 2.2466x; 1.1379x over previous
import functools

import jax
import jax.numpy as jnp
from jax import lax
from jax.experimental import pallas as pl
from jax.experimental.pallas import tpu as pltpu
from jax.experimental.pallas import tpu_sc as plsc

N = 10000
E = 160000
F = 128
NB = 8
SHD = 9
NSPEC = 10
CORR = 3
AVG = 16.0

NC = 2
NS = 16
NW = NC * NS
EPT = 5120
E_PAD = NW * EPT
BATCH = 128
NBATCH = EPT // BATCH
N_PAD = 10240
RPT = N_PAD // NS

BE = 2048
BN = 1000
NCHUNK = 4
CH = E_PAD // NCHUNK
EPT_C = CH // NW
NBATCH_C = EPT_C // BATCH


def _gather_body(nf_hbm, snd_hbm, h_hbm, idx_v, rows0, rows1, gs0, gs1,
                 os0, os1):
    c = lax.axis_index("c")
    s = lax.axis_index("s")
    base = (c * NS + s) * EPT_C
    pltpu.sync_copy(snd_hbm.at[pl.ds(base, EPT_C)], idx_v)

    def gat(b, rows, sem):
        off = pl.multiple_of(b * BATCH, BATCH)
        pltpu.async_copy(nf_hbm.at[idx_v.at[pl.ds(off, BATCH)]], rows, sem)

    def out(b, rows, sem):
        off = pl.multiple_of(b * BATCH, BATCH)
        pltpu.async_copy(rows, h_hbm.at[pl.ds(base + off, BATCH)], sem)

    def wait_g(rows, sem):
        pltpu.make_async_copy(nf_hbm.at[idx_v.at[pl.ds(0, BATCH)]], rows,
                              sem).wait()

    def wait_o(rows, sem):
        pltpu.make_async_copy(rows, h_hbm.at[pl.ds(base, BATCH)], sem).wait()

    gat(0, rows0, gs0)

    def body(jj, carry):
        b0 = jj * 2

        @pl.when(jj > 0)
        def _():
            wait_o(rows1, os1)

        gat(b0 + 1, rows1, gs1)
        wait_g(rows0, gs0)
        out(b0, rows0, os0)

        @pl.when(jj < NBATCH_C // 2 - 1)
        def _():
            wait_o(rows0, os0)
            gat(b0 + 2, rows0, gs0)

        wait_g(rows1, gs1)
        out(b0 + 1, rows1, os1)
        return carry

    lax.fori_loop(0, NBATCH_C // 2, body, 0)
    wait_o(rows0, os0)
    wait_o(rows1, os1)


@functools.cache
def _gather():
    mesh = plsc.VectorSubcoreMesh(core_axis_name="c", subcore_axis_name="s")
    return pl.kernel(
        _gather_body,
        out_type=jax.ShapeDtypeStruct((CH, F), jnp.float32),
        mesh=mesh,
        scratch_types=[
            pltpu.VMEM((EPT_C,), jnp.int32),
            pltpu.VMEM((BATCH, F), jnp.float32),
            pltpu.VMEM((BATCH, F), jnp.float32),
            pltpu.SemaphoreType.DMA,
            pltpu.SemaphoreType.DMA,
            pltpu.SemaphoreType.DMA,
            pltpu.SemaphoreType.DMA,
        ],
    )


def _scatter_body(m_hbm, recv3_hbm, zeros_hbm, out_hbm, ridx_v, m0, m1,
                  ls0, ls1, ss0, ss1, acc_sh):
    c = lax.axis_index("c")
    s = lax.axis_index("s")
    wid = c * NS + s
    base = wid * EPT_C
    row0 = s * RPT
    pltpu.sync_copy(zeros_hbm.at[pl.ds(row0, RPT)], acc_sh.at[pl.ds(row0, RPT)])
    pltpu.sync_copy(recv3_hbm.at[wid], ridx_v)
    plsc.subcore_barrier()

    def load(b, buf, sem):
        off = pl.multiple_of(b * BATCH, BATCH)
        pltpu.async_copy(m_hbm.at[pl.ds(base + off, BATCH)], buf, sem)

    def scat(b, buf, sem):
        pltpu.async_copy(buf, acc_sh.at[ridx_v.at[b]], sem, add=True)

    def wait_l(buf, sem):
        pltpu.make_async_copy(m_hbm.at[pl.ds(base, BATCH)], buf, sem).wait()

    def wait_s(buf, sem):
        pltpu.make_async_copy(buf, acc_sh.at[ridx_v.at[0]], sem).wait()

    load(0, m0, ls0)

    def body(jj, carry):
        b0 = jj * 2

        @pl.when(jj > 0)
        def _():
            wait_s(m1, ss1)

        load(b0 + 1, m1, ls1)
        wait_l(m0, ls0)
        scat(b0, m0, ss0)

        @pl.when(jj < NBATCH_C // 2 - 1)
        def _():
            wait_s(m0, ss0)
            load(b0 + 2, m0, ls0)

        wait_l(m1, ls1)
        scat(b0 + 1, m1, ss1)
        return carry

    lax.fori_loop(0, NBATCH_C // 2, body, 0)
    wait_s(m0, ss0)
    wait_s(m1, ss1)
    plsc.subcore_barrier()
    pltpu.sync_copy(acc_sh.at[pl.ds(row0, RPT)], out_hbm.at[c, pl.ds(row0, RPT)])


@functools.cache
def _scatter():
    mesh = plsc.VectorSubcoreMesh(core_axis_name="c", subcore_axis_name="s")
    return pl.kernel(
        _scatter_body,
        out_type=jax.ShapeDtypeStruct((NC, N_PAD, F), jnp.float32),
        mesh=mesh,
        scratch_types=[
            pltpu.VMEM((NBATCH_C, BATCH), jnp.int32),
            pltpu.VMEM((BATCH, F), jnp.float32),
            pltpu.VMEM((BATCH, F), jnp.float32),
            pltpu.SemaphoreType.DMA,
            pltpu.SemaphoreType.DMA,
            pltpu.SemaphoreType.DMA,
            pltpu.SemaphoreType.DMA,
            pltpu.VMEM_SHARED((N_PAD, F), jnp.float32),
        ],
    )


def _edge_body(vect_ref, radt_ref, h_ref, wr1t_ref, br1_ref, wr2t_ref,
               br2_ref, exp_ref, wlin_ref, m_ref):
    vt = vect_ref[...]
    r = jnp.sqrt(jnp.sum(vt * vt, axis=0, keepdims=True)) + 1e-8
    ut = vt / r
    x, y, z = ut[0:1, :], ut[1:2, :], ut[2:3, :]
    yht = jnp.concatenate([
        jnp.ones_like(x), x, y, z,
        x * y, y * z, 3.0 * z * z - 1.0, x * z, x * x - y * y,
    ], axis=0)
    rht = jnp.dot(wr1t_ref[...], radt_ref[...],
                  preferred_element_type=jnp.float32) + br1_ref[...]
    rht = rht * jax.nn.sigmoid(rht)
    rwt = jnp.dot(wr2t_ref[...], rht,
                  preferred_element_type=jnp.float32) + br2_ref[...]
    cct = yht * rwt
    cc = cct.astype(jnp.bfloat16).T
    h = h_ref[...].astype(jnp.bfloat16)
    x = jnp.concatenate([h * cc[:, lm:lm + 1] for lm in range(SHD)],
                        axis=1)
    acc = jnp.dot(x, wlin_ref[...], preferred_element_type=jnp.float32)
    m_ref[...] = acc * (1.0 / jnp.sqrt(AVG))


def _edge_call(vec_p, rad_p, h, wr1, br1, wr2, br2, wlin):
    grid = CH // BE
    exp9 = jnp.kron(jnp.eye(SHD, dtype=jnp.bfloat16),
                    jnp.ones((1, F), jnp.bfloat16))
    return pl.pallas_call(
        _edge_body,
        grid=(grid,),
        in_specs=[
            pl.BlockSpec((3, BE), lambda i: (0, i)),
            pl.BlockSpec((NB, BE), lambda i: (0, i)),
            pl.BlockSpec((BE, F), lambda i: (i, 0)),
            pl.BlockSpec((64, NB), lambda i: (0, 0)),
            pl.BlockSpec((64, 1), lambda i: (0, 0)),
            pl.BlockSpec((SHD, 64), lambda i: (0, 0)),
            pl.BlockSpec((SHD, 1), lambda i: (0, 0)),
            pl.BlockSpec((SHD, SHD * F), lambda i: (0, 0)),
            pl.BlockSpec((SHD * F, F), lambda i: (0, 0)),
        ],
        out_specs=pl.BlockSpec((BE, F), lambda i: (i, 0)),
        out_shape=jax.ShapeDtypeStruct((CH, F), jnp.float32),
    )(vec_p.T, rad_p.T, h, wr1.T, br1.reshape(64, 1), wr2.T,
      br2.reshape(SHD, 1), exp9, wlin.astype(jnp.bfloat16))


def _node_body(p_ref, nf_ref, spec_ref, wskip_ref, wprod_ref, wpl_ref,
               wread_ref, out1_ref, feats_ref):
    agg = jnp.sum(p_ref[...], axis=0)
    spec = spec_ref[...]
    onehot = (spec == lax.broadcasted_iota(jnp.int32, (1, NSPEC), 1)
              ).astype(jnp.float32)
    nf = nf_ref[...]
    skip = jnp.zeros((BN, F), jnp.float32)
    for sp in range(NSPEC):
        skip = skip + onehot[:, sp:sp + 1] * jnp.dot(
            nf, wskip_ref[sp], preferred_element_type=jnp.float32)
    w = jnp.dot(onehot, wprod_ref[...],
                preferred_element_type=jnp.float32)
    pb = jnp.zeros((BN, F), jnp.float32)
    p = agg
    for nu in range(CORR):
        pb = pb + w[:, nu * F:(nu + 1) * F] * p
        p = p * agg
    feats = jnp.dot(pb, wpl_ref[...], preferred_element_type=jnp.float32) + skip
    feats_ref[...] = feats
    out1_ref[...] = jnp.dot(feats, wread_ref[...],
                            preferred_element_type=jnp.float32)


def _node_call(partials, nf, spec2, wskip, wprod2, wpl, wread):
    grid = N // BN
    return pl.pallas_call(
        _node_body,
        grid=(grid,),
        in_specs=[
            pl.BlockSpec((NCHUNK * NC, BN, F), lambda i: (0, i, 0)),
            pl.BlockSpec((BN, F), lambda i: (i, 0)),
            pl.BlockSpec((BN, 1), lambda i: (i, 0)),
            pl.BlockSpec((NSPEC, F, F), lambda i: (0, 0, 0)),
            pl.BlockSpec((NSPEC, CORR * F), lambda i: (0, 0)),
            pl.BlockSpec((F, F), lambda i: (0, 0)),
            pl.BlockSpec((F, 1), lambda i: (0, 0)),
        ],
        out_specs=[
            pl.BlockSpec((BN, 1), lambda i: (i, 0)),
            pl.BlockSpec((BN, F), lambda i: (i, 0)),
        ],
        out_shape=[
            jax.ShapeDtypeStruct((N, 1), jnp.float32),
            jax.ShapeDtypeStruct((N, F), jnp.float32),
        ],
    )(partials, nf, spec2, wskip, wprod2, wpl, wread)


def kernel(vectors, node_feats, node_specie, radial_embedding, senders,
           receivers, W_skip, Wr1, br1, Wr2, br2, W_lin, w_prod, W_prodlin,
           W_read):
    pad = E_PAD - E
    snd = jnp.concatenate(
        [senders.astype(jnp.int32), jnp.zeros((pad,), jnp.int32)])
    rcv = jnp.concatenate(
        [receivers.astype(jnp.int32), jnp.full((pad,), N, jnp.int32)])
    vec_p = jnp.concatenate([vectors, jnp.ones((pad, 3), jnp.float32)])
    rad_p = jnp.concatenate(
        [radial_embedding, jnp.zeros((pad, NB), jnp.float32)])
    zeros = jnp.zeros((N_PAD, F), jnp.float32)

    partials = []
    for k in range(NCHUNK):
        sl = slice(k * CH, (k + 1) * CH)
        h = _gather()(node_feats, snd[sl])
        m = _edge_call(vec_p[sl], rad_p[sl], h, Wr1, br1, Wr2, br2, W_lin)
        recv3 = rcv[sl].reshape(NW, NBATCH_C, BATCH)
        partials.append(_scatter()(m, recv3, zeros))
    out1, feats = _node_call(
        jnp.concatenate(partials, axis=0)[:, :N, :], node_feats,
        node_specie.reshape(N, 1).astype(jnp.int32), W_skip,
        w_prod.reshape(NSPEC, CORR * F), W_prodlin, W_read)
    return out1, feats
